# baseline (speedup 1.0000x reference)
import math

import jax
import jax.numpy as jnp
from jax import lax
from jax.experimental import pallas as pl
from jax.experimental.pallas import tpu as pltpu
from jax.experimental.pallas import tpu_sc as plsc

_NUM_LEVELS = 16
_FEATS = 2
_TABLE = 2 ** 19
_MASK = _TABLE - 1
_MIN_RES = 16
_MAX_RES = 512
_P1 = 1540863
_P2 = 1256879
_P3 = 1957123

_GROWTH = math.exp(math.log(_MAX_RES / _MIN_RES) / (_NUM_LEVELS - 1))
_RES = [int(math.floor(_MIN_RES * _GROWTH ** l + 1e-06)) for l in range(_NUM_LEVELS)]

_CORNERS = [(ox, oy, oz) for ox in (0, 1) for oy in (0, 1) for oz in (0, 1)]

_NC = 2
_NS = 16
_NW = _NC * _NS

_C = 1024
_G = _C // 16
_E = _C * 8
_SEG = _TABLE // _NS
_SUB = 4
_ESUB = 2 * _SEG // _SUB


def _vdup(v, idx):
    dn = lax.GatherDimensionNumbers(
        offset_dims=(), collapsed_slice_dims=(0,), start_index_map=(0,))
    return lax.gather(v, idx[:, None], dn, (1,),
                      mode=lax.GatherScatterMode.PROMISE_IN_BOUNDS)


def _hash_grid_sc(x2d, tab_flat, n_points):
    per_w = n_points // _NW
    n_chunks = per_w // _C

    mesh = plsc.VectorSubcoreMesh(core_axis_name="c", subcore_axis_name="s")

    def body(x_hbm, tab_hbm, res_hbm, out_hbm, x_v, res_v, idx_v, w_v, rows_v,
             outb0_v, outb1_v, pk_v, pko_v, spm_v, sem):
        wid = lax.axis_index("s") * _NC + lax.axis_index("c")
        wbase = wid * per_w
        sid = lax.axis_index("s")

        pltpu.sync_copy(res_hbm, res_v)
        resvec = res_v[...]

        def level_body(l, _carry):
            plsc.subcore_barrier()

            def stage_sub(s, _):
                eoff = l * (2 * _TABLE) + sid * (2 * _SEG) + s * _ESUB
                pltpu.sync_copy(tab_hbm.at[pl.ds(eoff, _ESUB)], pk_v)

                def pack_rb(b, _):
                    for t in range(8):
                        v0 = pk_v[pl.ds(b * 256 + t * 16, 16)]
                        v1 = pk_v[pl.ds(b * 256 + 128 + t * 16, 16)]
                        u0 = lax.bitcast_convert_type(v0, jnp.uint32)
                        u1 = lax.bitcast_convert_type(v1, jnp.uint32)
                        half = jnp.uint32(0x8000)
                        hi = jnp.uint32(0xFFFF0000)
                        p = ((u0 + half) >> 16) | ((u1 + half) & hi)
                        pko_v[pl.ds(b * 128 + t * 16, 16)] = p
                    return 0

                lax.fori_loop(0, _ESUB // 256, pack_rb, 0)
                soff = sid * _SEG + s * (_ESUB // 2)
                pltpu.sync_copy(pko_v, spm_v.at[pl.ds(soff, _ESUB // 2)])
                return 0

            lax.fori_loop(0, _SUB, stage_sub, 0)
            plsc.subcore_barrier()

            resb = _vdup(resvec, jnp.full((16,), l, dtype=jnp.int32))

            def phase1(c):
                par = c & 1
                ib = par * _E

                def index_group(g, _):
                    col = (c & 1) * _C + g * 16
                    xv = x_v[0, pl.ds(col, 16)]
                    yv = x_v[1, pl.ds(col, 16)]
                    zv = x_v[2, pl.ds(col, 16)]
                    px = xv * resb
                    py = yv * resb
                    pz = zv * resb
                    ix0 = px.astype(jnp.int32)
                    iy0 = py.astype(jnp.int32)
                    iz0 = pz.astype(jnp.int32)
                    fx = px - ix0.astype(jnp.float32)
                    fy = py - iy0.astype(jnp.float32)
                    fz = pz - iz0.astype(jnp.float32)
                    hx = (ix0 * _P1, ix0 * _P1 + _P1)
                    hy = (iy0 * _P2, iy0 * _P2 + _P2)
                    hz = (iz0 * _P3, iz0 * _P3 + _P3)
                    wx = (1.0 - fx, fx)
                    wy = (1.0 - fy, fy)
                    wz = (1.0 - fz, fz)
                    for j, (ox, oy, oz) in enumerate(_CORNERS):
                        h = (hx[ox] ^ hy[oy]) ^ hz[oz]
                        off = ib + (g * 8 + j) * 16
                        idx_v[pl.ds(off, 16)] = h & _MASK
                        w_v[pl.ds(off, 16)] = (wx[ox] * wy[oy]) * wz[oz]
                    return 0

                lax.fori_loop(0, _G, index_group, 0, unroll=4)

            def gather_refs(c):
                par = c & 1
                return (spm_v.at[idx_v.at[pl.ds(par * _E, _E)]],
                        rows_v.at[pl.ds(par * _E, _E)])

            def fire(c):
                src, dst = gather_refs(c)
                pltpu.async_copy(src, dst, sem)

            def wait_g(c):
                src, dst = gather_refs(c)
                pltpu.make_async_copy(src, dst, sem).wait()

            def phase3(c):
                par = c & 1
                ib = par * _E
                hi = jnp.uint32(0xFFFF0000)

                def acc_group(g, _):
                    acc0 = None
                    acc1 = None
                    for j in range(8):
                        off = ib + (g * 8 + j) * 16
                        u = rows_v[pl.ds(off, 16)]
                        f0 = lax.bitcast_convert_type(u << 16, jnp.float32)
                        f1 = lax.bitcast_convert_type(u & hi, jnp.float32)
                        wj = w_v[pl.ds(off, 16)]
                        if acc0 is None:
                            acc0 = wj * f0
                            acc1 = wj * f1
                        else:
                            acc0 = acc0 + wj * f0
                            acc1 = acc1 + wj * f1
                    outb0_v[pl.ds(g * 16, 16)] = acc0
                    outb1_v[pl.ds(g * 16, 16)] = acc1
                    return 0

                lax.fori_loop(0, _G, acc_group, 0, unroll=4)
                pbase = wbase + c * _C
                obase = (2 * l) * n_points + pbase
                pltpu.sync_copy(outb0_v, out_hbm.at[pl.ds(obase, _C)])
                obase1 = (2 * l + 1) * n_points + pbase
                pltpu.sync_copy(outb1_v, out_hbm.at[pl.ds(obase1, _C)])

            def chunk_body(c, carry):
                pl.when((c & 1) == 0)(lambda: pltpu.sync_copy(
                    x_hbm.at[:, pl.ds(wbase + (c >> 1) * (2 * _C), 2 * _C)], x_v))
                phase1(c)
                pl.when(c > 0)(lambda: wait_g(c - 1))
                fire(c)
                pl.when(c > 0)(lambda: phase3(c - 1))
                return carry

            lax.fori_loop(0, n_chunks, chunk_body, 0)
            wait_g(n_chunks - 1)
            phase3(n_chunks - 1)
            return _carry

        lax.fori_loop(0, _NUM_LEVELS, level_body, 0)

    kern = pl.kernel(
        body,
        out_type=jax.ShapeDtypeStruct((_NUM_LEVELS * _FEATS * n_points,),
                                      jnp.float32),
        mesh=mesh,
        scratch_types=[
            pltpu.VMEM((3, 2 * _C), jnp.float32),
            pltpu.VMEM((16,), jnp.float32),
            pltpu.VMEM((2 * _E,), jnp.int32),
            pltpu.VMEM((2 * _E,), jnp.float32),
            pltpu.VMEM((2 * _E,), jnp.uint32),
            pltpu.VMEM((_C,), jnp.float32),
            pltpu.VMEM((_C,), jnp.float32),
            pltpu.VMEM((_ESUB,), jnp.float32),
            pltpu.VMEM((_ESUB // 2,), jnp.uint32),
            pltpu.VMEM_SHARED((_TABLE,), jnp.uint32),
            pltpu.SemaphoreType.DMA,
        ],
        compiler_params=pltpu.CompilerParams(needs_layout_passes=False),
    )
    res_arr = jnp.asarray([float(r) for r in _RES], dtype=jnp.float32)
    return kern(x2d, tab_flat, res_arr)


def kernel(x01, tables):
    n = x01.shape[0]
    x2d = x01.T
    tab_flat = tables.reshape(_NUM_LEVELS, _TABLE // 128, 128, _FEATS)
    tab_flat = tab_flat.transpose(0, 1, 3, 2).reshape(-1)
    out = _hash_grid_sc(x2d, tab_flat, n)
    return out.reshape(_NUM_LEVELS * _FEATS, n).T


# speedup vs baseline: 1.0125x; 1.0125x over previous
import math

import jax
import jax.numpy as jnp
from jax import lax
from jax.experimental import pallas as pl
from jax.experimental.pallas import tpu as pltpu
from jax.experimental.pallas import tpu_sc as plsc

_NUM_LEVELS = 16
_FEATS = 2
_TABLE = 2 ** 19
_MASK = _TABLE - 1
_MIN_RES = 16
_MAX_RES = 512
_P1 = 1540863
_P2 = 1256879
_P3 = 1957123

_GROWTH = math.exp(math.log(_MAX_RES / _MIN_RES) / (_NUM_LEVELS - 1))
_RES = [int(math.floor(_MIN_RES * _GROWTH ** l + 1e-06)) for l in range(_NUM_LEVELS)]

_CORNERS = [(ox, oy, oz) for ox in (0, 1) for oy in (0, 1) for oz in (0, 1)]

_NC = 2
_NS = 16
_NW = _NC * _NS

_C = 1024
_G = _C // 16
_E = _C * 8
_SEG = _TABLE // _NS
_SUB = 4
_ESUB = 2 * _SEG // _SUB


def _vdup(v, idx):
    dn = lax.GatherDimensionNumbers(
        offset_dims=(), collapsed_slice_dims=(0,), start_index_map=(0,))
    return lax.gather(v, idx[:, None], dn, (1,),
                      mode=lax.GatherScatterMode.PROMISE_IN_BOUNDS)


def _hash_grid_sc(x2d, tab_flat, n_points):
    per_w = n_points // _NW
    n_chunks = per_w // _C

    mesh = plsc.VectorSubcoreMesh(core_axis_name="c", subcore_axis_name="s")

    def body(x_hbm, tab_hbm, res_hbm, out_hbm, x_v, res_v, idx_v, w_v, rows_v,
             outb0_v, outb1_v, pk_v, pko_v, spm_v, sem):
        wid = lax.axis_index("s") * _NC + lax.axis_index("c")
        wbase = wid * per_w
        sid = lax.axis_index("s")

        pltpu.sync_copy(res_hbm, res_v)
        resvec = res_v[...]

        def level_body(l, _carry):
            plsc.subcore_barrier()

            def stage_sub(s, _):
                eoff = l * (2 * _TABLE) + sid * (2 * _SEG) + s * _ESUB
                pltpu.sync_copy(tab_hbm.at[pl.ds(eoff, _ESUB)], pk_v)

                def pack_rb(b, _):
                    for t in range(8):
                        v0 = pk_v[pl.ds(b * 256 + t * 16, 16)]
                        v1 = pk_v[pl.ds(b * 256 + 128 + t * 16, 16)]
                        u0 = lax.bitcast_convert_type(v0, jnp.uint32)
                        u1 = lax.bitcast_convert_type(v1, jnp.uint32)
                        half = jnp.uint32(0x8000)
                        hi = jnp.uint32(0xFFFF0000)
                        p = ((u0 + half) >> 16) | ((u1 + half) & hi)
                        pko_v[pl.ds(b * 128 + t * 16, 16)] = p
                    return 0

                lax.fori_loop(0, _ESUB // 256, pack_rb, 0)
                soff = sid * _SEG + s * (_ESUB // 2)
                pltpu.sync_copy(pko_v, spm_v.at[pl.ds(soff, _ESUB // 2)])
                return 0

            lax.fori_loop(0, _SUB, stage_sub, 0)
            plsc.subcore_barrier()

            resb = _vdup(resvec, jnp.full((16,), l, dtype=jnp.int32))

            def phase1(c):
                par = c & 1
                ib = par * _E

                def index_group(g, _):
                    col = (c & 1) * _C + g * 16
                    xv = x_v[0, pl.ds(col, 16)]
                    yv = x_v[1, pl.ds(col, 16)]
                    zv = x_v[2, pl.ds(col, 16)]
                    px = xv * resb
                    py = yv * resb
                    pz = zv * resb
                    ix0 = px.astype(jnp.int32)
                    iy0 = py.astype(jnp.int32)
                    iz0 = pz.astype(jnp.int32)
                    fx = px - ix0.astype(jnp.float32)
                    fy = py - iy0.astype(jnp.float32)
                    fz = pz - iz0.astype(jnp.float32)
                    hx = (ix0 * _P1, ix0 * _P1 + _P1)
                    hy = (iy0 * _P2, iy0 * _P2 + _P2)
                    hz = (iz0 * _P3, iz0 * _P3 + _P3)
                    wx = (1.0 - fx, fx)
                    wy = (1.0 - fy, fy)
                    wz = (1.0 - fz, fz)
                    for j, (ox, oy, oz) in enumerate(_CORNERS):
                        h = (hx[ox] ^ hy[oy]) ^ hz[oz]
                        off = ib + (g * 8 + j) * 16
                        idx_v[pl.ds(off, 16)] = h & _MASK
                        w_v[pl.ds(off, 16)] = (wx[ox] * wy[oy]) * wz[oz]
                    return 0

                lax.fori_loop(0, _G, index_group, 0, unroll=2)

            def gather_refs(c):
                par = c & 1
                return (spm_v.at[idx_v.at[pl.ds(par * _E, _E)]],
                        rows_v.at[pl.ds(par * _E, _E)])

            def fire(c):
                src, dst = gather_refs(c)
                pltpu.async_copy(src, dst, sem)

            def wait_g(c):
                src, dst = gather_refs(c)
                pltpu.make_async_copy(src, dst, sem).wait()

            def phase3(c):
                par = c & 1
                ib = par * _E
                hi = jnp.uint32(0xFFFF0000)

                def acc_group(g, _):
                    acc0 = None
                    acc1 = None
                    for j in range(8):
                        off = ib + (g * 8 + j) * 16
                        u = rows_v[pl.ds(off, 16)]
                        f0 = lax.bitcast_convert_type(u << 16, jnp.float32)
                        f1 = lax.bitcast_convert_type(u & hi, jnp.float32)
                        wj = w_v[pl.ds(off, 16)]
                        if acc0 is None:
                            acc0 = wj * f0
                            acc1 = wj * f1
                        else:
                            acc0 = acc0 + wj * f0
                            acc1 = acc1 + wj * f1
                    outb0_v[pl.ds(g * 16, 16)] = acc0
                    outb1_v[pl.ds(g * 16, 16)] = acc1
                    return 0

                lax.fori_loop(0, _G, acc_group, 0, unroll=2)
                pbase = wbase + c * _C
                obase = (2 * l) * n_points + pbase
                pltpu.sync_copy(outb0_v, out_hbm.at[pl.ds(obase, _C)])
                obase1 = (2 * l + 1) * n_points + pbase
                pltpu.sync_copy(outb1_v, out_hbm.at[pl.ds(obase1, _C)])

            def chunk_body(c, carry):
                pl.when((c & 1) == 0)(lambda: pltpu.sync_copy(
                    x_hbm.at[:, pl.ds(wbase + (c >> 1) * (2 * _C), 2 * _C)], x_v))
                phase1(c)
                pl.when(c > 0)(lambda: wait_g(c - 1))
                fire(c)
                pl.when(c > 0)(lambda: phase3(c - 1))
                return carry

            lax.fori_loop(0, n_chunks, chunk_body, 0)
            wait_g(n_chunks - 1)
            phase3(n_chunks - 1)
            return _carry

        lax.fori_loop(0, _NUM_LEVELS, level_body, 0)

    kern = pl.kernel(
        body,
        out_type=jax.ShapeDtypeStruct((_NUM_LEVELS * _FEATS * n_points,),
                                      jnp.float32),
        mesh=mesh,
        scratch_types=[
            pltpu.VMEM((3, 2 * _C), jnp.float32),
            pltpu.VMEM((16,), jnp.float32),
            pltpu.VMEM((2 * _E,), jnp.int32),
            pltpu.VMEM((2 * _E,), jnp.float32),
            pltpu.VMEM((2 * _E,), jnp.uint32),
            pltpu.VMEM((_C,), jnp.float32),
            pltpu.VMEM((_C,), jnp.float32),
            pltpu.VMEM((_ESUB,), jnp.float32),
            pltpu.VMEM((_ESUB // 2,), jnp.uint32),
            pltpu.VMEM_SHARED((_TABLE,), jnp.uint32),
            pltpu.SemaphoreType.DMA,
        ],
        compiler_params=pltpu.CompilerParams(needs_layout_passes=False),
    )
    res_arr = jnp.asarray([float(r) for r in _RES], dtype=jnp.float32)
    return kern(x2d, tab_flat, res_arr)


def kernel(x01, tables):
    n = x01.shape[0]
    x2d = x01.T
    tab_flat = tables.reshape(_NUM_LEVELS, _TABLE // 128, 128, _FEATS)
    tab_flat = tab_flat.transpose(0, 1, 3, 2).reshape(-1)
    out = _hash_grid_sc(x2d, tab_flat, n)
    return out.reshape(_NUM_LEVELS * _FEATS, n).T
